# Initial kernel scaffold; baseline (speedup 1.0000x reference)
import jax
import jax.numpy as jnp
from jax.experimental import pallas as pl


def kernel(x, table):
    raise NotImplementedError("write your pallas kernel here")


# speedup vs baseline: 1.0233x; 1.0233x over previous
import functools

import jax
import jax.numpy as jnp
from jax import lax
from jax.experimental import pallas as pl
from jax.experimental.pallas import tpu as pltpu
from jax.experimental.pallas import tpu_sc as plsc

_VOC_LEN = 1000000
_D = 32
_SCALE = float(_D) ** 0.5

_info = plsc.get_sparse_core_info()
_NC, _NS, _L = _info.num_cores, _info.num_subcores, _info.num_lanes
_NW = _NC * _NS


def _make_lookup(B: int, C: int):
    assert B % (_NW * C) == 0
    b_per_w = B // _NW
    n_chunks = b_per_w // C
    mesh = plsc.VectorSubcoreMesh(core_axis_name="c", subcore_axis_name="s")

    @functools.partial(
        pl.kernel,
        mesh=mesh,
        compiler_params=pltpu.CompilerParams(use_tc_tiling_on_sc=False),
        out_type=jax.ShapeDtypeStruct((B, _D), jnp.float32),
        scratch_types=[
            pltpu.VMEM((C,), jnp.int32),
            pltpu.VMEM((C, _D), jnp.float32),
            pltpu.SemaphoreType.DMA,
        ],
    )
    def lookup(x_hbm, tab_hbm, out_hbm, idx_v, rows_v, sem):
        wid = lax.axis_index("s") * _NC + lax.axis_index("c")
        w_base = wid * b_per_w

        def chunk_body(i, carry):
            base = w_base + i * C
            pltpu.sync_copy(x_hbm.at[pl.ds(base, C)], idx_v)
            pltpu.async_copy(tab_hbm.at[idx_v], rows_v, sem).wait()

            def scale_row(j, carry2):
                rows_v[j, pl.ds(0, _L)] = rows_v[j, pl.ds(0, _L)] * _SCALE
                rows_v[j, pl.ds(_L, _L)] = rows_v[j, pl.ds(_L, _L)] * _SCALE
                return carry2

            lax.fori_loop(0, C, scale_row, 0, unroll=4)
            pltpu.sync_copy(rows_v, out_hbm.at[pl.ds(base, C)])
            return carry

        lax.fori_loop(0, n_chunks, chunk_body, 0)

    return lookup


_B = 16384 * 50
_lookup = _make_lookup(_B, 1600)


def kernel(x, table):
    xf = x.reshape(-1).astype(jnp.int32)
    out = _lookup(xf, table)
    return out.reshape(x.shape + (_D,))


# speedup vs baseline: 1.3455x; 1.3149x over previous
import functools

import jax
import jax.numpy as jnp
from jax import lax
from jax.experimental import pallas as pl
from jax.experimental.pallas import tpu as pltpu
from jax.experimental.pallas import tpu_sc as plsc

_VOC_LEN = 1000000
_D = 32
_S = 50
_SCALE = float(_D) ** 0.5

_info = plsc.get_sparse_core_info()
_NC, _NS, _L = _info.num_cores, _info.num_subcores, _info.num_lanes
_NW = _NC * _NS


def _make_lookup(NB: int, R: int):
    b_per_w = NB // _NW
    assert NB % _NW == 0 and b_per_w % R == 0
    n_chunks = b_per_w // R
    C = R * _S
    mesh = plsc.VectorSubcoreMesh(core_axis_name="c", subcore_axis_name="s")

    @functools.partial(
        pl.kernel,
        mesh=mesh,
        compiler_params=pltpu.CompilerParams(use_tc_tiling_on_sc=False),
        out_type=jax.ShapeDtypeStruct((NB, _S, _D), jnp.float32),
        scratch_types=[
            pltpu.VMEM((C,), jnp.int32),
            pltpu.VMEM((C, _D), jnp.float32),
            pltpu.VMEM((R, _S, _D), jnp.float32),
            pltpu.SemaphoreType.DMA,
        ],
    )
    def lookup(x_hbm, tab_hbm, out_hbm, idx_v, rows_v, stage_v, sem):
        wid = lax.axis_index("s") * _NC + lax.axis_index("c")
        g_base = wid * b_per_w * _S

        def chunk_body(i, carry):
            pltpu.sync_copy(x_hbm.at[pl.ds(g_base + i * C, C)], idx_v)
            pltpu.async_copy(tab_hbm.at[idx_v], rows_v, sem).wait()

            def row_body(r, carry2):
                def tok_body(s, carry3):
                    g = r * _S + s
                    stage_v[r, s, pl.ds(0, _L)] = rows_v[g, pl.ds(0, _L)] * _SCALE
                    stage_v[r, s, pl.ds(_L, _L)] = rows_v[g, pl.ds(_L, _L)] * _SCALE
                    return carry3

                return lax.fori_loop(0, _S, tok_body, carry2, unroll=5)

            lax.fori_loop(0, R, row_body, 0)
            pltpu.sync_copy(stage_v, out_hbm.at[pl.ds(wid * b_per_w + i * R, R)])
            return carry

        lax.fori_loop(0, n_chunks, chunk_body, 0)

    return lookup


_NB = 16384
_lookup = _make_lookup(_NB, 16)


def kernel(x, table):
    xf = x.reshape(-1).astype(jnp.int32)
    return _lookup(xf, table)


# speedup vs baseline: 1.7063x; 1.2681x over previous
import functools

import jax
import jax.numpy as jnp
from jax import lax
from jax.experimental import pallas as pl
from jax.experimental.pallas import tpu as pltpu
from jax.experimental.pallas import tpu_sc as plsc

_VOC_LEN = 1000000
_D = 32
_S = 50
_SCALE = float(_D) ** 0.5

_info = plsc.get_sparse_core_info()
_NC, _NS, _L = _info.num_cores, _info.num_subcores, _info.num_lanes
_NW = _NC * _NS
_NBUF = 2


def _make_lookup(NB: int, R: int):
    b_per_w = NB // _NW
    assert NB % _NW == 0 and b_per_w % R == 0
    n_chunks = b_per_w // R
    assert n_chunks % _NBUF == 0 and n_chunks >= 2 * _NBUF
    C = R * _S
    mesh = plsc.VectorSubcoreMesh(core_axis_name="c", subcore_axis_name="s")

    @functools.partial(
        pl.kernel,
        mesh=mesh,
        compiler_params=pltpu.CompilerParams(use_tc_tiling_on_sc=False),
        out_type=jax.ShapeDtypeStruct((NB, _S, _D), jnp.float32),
        scratch_types=[
            [pltpu.VMEM((C,), jnp.int32) for _ in range(_NBUF)],
            [pltpu.VMEM((C, _D), jnp.float32) for _ in range(_NBUF)],
            [pltpu.SemaphoreType.DMA for _ in range(_NBUF)],
            [pltpu.SemaphoreType.DMA for _ in range(_NBUF)],
        ],
    )
    def lookup(x_hbm, tab_hbm, out_hbm, idx_v, rows_v, gsem, wsem):
        wid = lax.axis_index("s") * _NC + lax.axis_index("c")
        g_base = wid * b_per_w * _S
        row_base = wid * b_per_w

        def load_chunk(j, k):
            pltpu.sync_copy(x_hbm.at[pl.ds(g_base + j * C, C)], idx_v[k])
            pltpu.make_async_copy(tab_hbm.at[idx_v[k]], rows_v[k], gsem[k]).start()

        def drain_gather(k):
            pltpu.make_async_copy(tab_hbm.at[idx_v[k]], rows_v[k], gsem[k]).wait()

        def scale_chunk(k):
            def body(r, carry):
                rows_v[k][r, pl.ds(0, _L)] = rows_v[k][r, pl.ds(0, _L)] * _SCALE
                rows_v[k][r, pl.ds(_L, _L)] = rows_v[k][r, pl.ds(_L, _L)] * _SCALE
                return carry

            lax.fori_loop(0, C, body, 0, unroll=8)

        def write_chunk(j, k):
            b0 = row_base + j * R
            for r in range(R):
                pltpu.make_async_copy(
                    rows_v[k].at[pl.ds(r * _S, _S), :], out_hbm.at[b0 + r], wsem[k]
                ).start()

        def drain_writes(j, k):
            b0 = row_base + j * R
            for r in range(R):
                pltpu.make_async_copy(
                    rows_v[k].at[pl.ds(r * _S, _S), :], out_hbm.at[b0 + r], wsem[k]
                ).wait()

        for k in range(_NBUF):
            load_chunk(k, k)

        def group_body(g, carry):
            for k in range(_NBUF):
                j = g * _NBUF + k
                drain_gather(k)
                scale_chunk(k)
                write_chunk(j, k)
                jn = j + _NBUF

                @pl.when(jn < n_chunks)
                def _():
                    pltpu.sync_copy(x_hbm.at[pl.ds(g_base + jn * C, C)], idx_v[k])
                    drain_writes(j, k)
                    pltpu.make_async_copy(
                        tab_hbm.at[idx_v[k]], rows_v[k], gsem[k]
                    ).start()

            return carry

        lax.fori_loop(0, n_chunks // _NBUF, group_body, 0)

        for k in range(_NBUF):
            drain_writes(n_chunks - _NBUF + k, k)

    return lookup


_NB = 16384
_lookup = _make_lookup(_NB, 32)


def kernel(x, table):
    xf = x.reshape(-1).astype(jnp.int32)
    return _lookup(xf, table)
